# baseline (speedup 1.0000x reference)
import jax
import jax.numpy as jnp
from jax import lax
from jax.experimental import pallas as pl
from jax.experimental.pallas import tpu as pltpu
from jax.experimental.pallas import tpu_sc as plsc

B, H, W, L = 8, 90, 256, 256
WL = W * L
HS = 48
NS, LANES = 16, 16
NQ = 2
CH = WL // (2 * NQ)
WQ = W // (2 * NQ)
NV = CH // LANES
HBT = 16
NHT = HS // HBT


def _tc_body(cs_ref, csp_ref, m_ref, out_ref, acc_ref):
    b = pl.program_id(0)
    k = pl.program_id(1)

    @pl.when((b == 0) & (k == 0))
    def _init():
        acc_ref[...] = jnp.zeros_like(acc_ref)

    vacc = jnp.zeros((HBT, L), jnp.float32)
    for w in range(W):
        a = cs_ref[0, :, pl.ds(w * L, L)]
        p = csp_ref[0, :, w, :]
        m = m_ref[0, w, :]
        a = jnp.where(jnp.isnan(a), 0.0, a)
        d = a - jnp.where(m > 0.0, p, 0.0)
        vacc = vacc + d * d
    acc_ref[...] += vacc

    @pl.when((b == B - 1) & (k == NHT - 1))
    def _fin():
        out_ref[0, 0] = jnp.sum(acc_ref[...])


def _tc_call(cs, cs_p, m):
    return pl.pallas_call(
        _tc_body,
        grid=(B, NHT),
        in_specs=[
            pl.BlockSpec((1, HBT, WL), lambda b, k: (b, k, 0)),
            pl.BlockSpec((1, HBT, W, L), lambda b, k: (b, k, 0, 0)),
            pl.BlockSpec((1, W, L), lambda b, k: (b, 0, 0)),
        ],
        out_specs=pl.BlockSpec(memory_space=pltpu.SMEM),
        out_shape=jax.ShapeDtypeStruct((1, 1), jnp.float32),
        scratch_shapes=[pltpu.VMEM((HBT, L), jnp.float32)],
    )(cs, cs_p, m)


def _make_sc_body(cid):
    def _sc_body(cs_hbm, csp_hbm, m_hbm, out_hbm,
                 mbuf, a0, a1, p0, p1, obuf, sa0, sp0, sa1, sp1):
        s = lax.axis_index("s")
        b = s // NQ
        q = 2 * (s % NQ) + cid
        qoff = q * CH
        w0 = q * WQ

        pltpu.sync_copy(m_hbm.at[b, pl.ds(w0, WQ), :], mbuf)

        def _binm(i, carry):
            w = i >> 4
            g = i & 15
            m = mbuf[w, pl.ds(g * LANES, LANES)]
            mbuf[w, pl.ds(g * LANES, LANES)] = jnp.where(m > 0.0, 1.0, 0.0)
            return carry
        lax.fori_loop(0, NV, _binm, 0)

        pltpu.async_copy(cs_hbm.at[b, HS, pl.ds(qoff, CH)], a0, sa0)
        pltpu.async_copy(csp_hbm.at[b, HS, pl.ds(w0, WQ), :], p0, sp0)
        pltpu.async_copy(cs_hbm.at[b, HS + 1, pl.ds(qoff, CH)], a1, sa1)
        pltpu.async_copy(csp_hbm.at[b, HS + 1, pl.ds(w0, WQ), :], p1, sp1)

        def _chunk(abuf, pbuf, acc):
            def _inner(i, acc):
                w = i >> 4
                g = i & 15
                a = abuf[pl.ds(i * LANES, LANES)]
                p = pbuf[w, pl.ds(g * LANES, LANES)]
                m = mbuf[w, pl.ds(g * LANES, LANES)]
                a = jnp.where(jnp.isnan(a), 0.0, a)
                d = a - p * m
                return acc + d * d
            return lax.fori_loop(0, NV, _inner, acc, unroll=8)

        def _outer(k, acc):
            h0 = HS + 2 * k
            pltpu.make_async_copy(cs_hbm.at[b, h0, pl.ds(qoff, CH)], a0, sa0).wait()
            pltpu.make_async_copy(csp_hbm.at[b, h0, pl.ds(w0, WQ), :], p0, sp0).wait()
            acc = _chunk(a0, p0, acc)

            @pl.when(h0 + 2 < H)
            def _():
                pltpu.async_copy(cs_hbm.at[b, h0 + 2, pl.ds(qoff, CH)], a0, sa0)
                pltpu.async_copy(csp_hbm.at[b, h0 + 2, pl.ds(w0, WQ), :], p0, sp0)

            pltpu.make_async_copy(cs_hbm.at[b, h0 + 1, pl.ds(qoff, CH)], a1, sa1).wait()
            pltpu.make_async_copy(csp_hbm.at[b, h0 + 1, pl.ds(w0, WQ), :], p1, sp1).wait()
            acc = _chunk(a1, p1, acc)

            @pl.when(h0 + 3 < H)
            def _():
                pltpu.async_copy(cs_hbm.at[b, h0 + 3, pl.ds(qoff, CH)], a1, sa1)
                pltpu.async_copy(csp_hbm.at[b, h0 + 3, pl.ds(w0, WQ), :], p1, sp1)
            return acc

        acc = lax.fori_loop(0, (H - HS) // 2, _outer,
                            jnp.zeros((LANES,), jnp.float32))
        obuf[...] = acc
        pltpu.sync_copy(obuf, out_hbm.at[s])
    return _sc_body


def _make_sc_call(cid):
    mesh = plsc.VectorSubcoreMesh(core_axis_name="c", subcore_axis_name="s",
                                  num_cores=1)
    return pl.kernel(
        _make_sc_body(cid),
        out_type=jax.ShapeDtypeStruct((NS, LANES), jnp.float32),
        mesh=mesh,
        scratch_types=[
            pltpu.VMEM((WQ, L), jnp.float32),
            pltpu.VMEM((CH,), jnp.float32),
            pltpu.VMEM((CH,), jnp.float32),
            pltpu.VMEM((WQ, L), jnp.float32),
            pltpu.VMEM((WQ, L), jnp.float32),
            pltpu.VMEM((LANES,), jnp.float32),
            pltpu.SemaphoreType.DMA,
            pltpu.SemaphoreType.DMA,
            pltpu.SemaphoreType.DMA,
            pltpu.SemaphoreType.DMA,
        ],
    )


_sc_call0 = _make_sc_call(0)
_sc_call1 = _make_sc_call(1)


def kernel(cs, cs_p, overpass_mask):
    sc_p0 = _sc_call0(cs, cs_p, overpass_mask)
    sc_p1 = _sc_call1(cs, cs_p, overpass_mask)
    tc_sum = _tc_call(cs, cs_p, overpass_mask)
    total = tc_sum[0, 0] + jnp.sum(sc_p0) + jnp.sum(sc_p1)
    return total / jnp.float32(B * H * WL)


# speedup vs baseline: 1.1935x; 1.1935x over previous
import jax
import jax.numpy as jnp
from jax import lax
from jax.experimental import pallas as pl
from jax.experimental.pallas import tpu as pltpu
from jax.experimental.pallas import tpu_sc as plsc

B, H, W, L = 8, 90, 256, 256
WL = W * L
HS = 56
NC, NS, LANES = 2, 16, 16
NW = NC * NS
NQ = NW // B
CH = WL // NQ
WQ = W // NQ
NV = CH // LANES
HBT = 8
NHT = HS // HBT


def _tc_body(cs_ref, csp_ref, m_ref, out_ref, acc_ref):
    b = pl.program_id(0)
    k = pl.program_id(1)

    @pl.when((b == 0) & (k == 0))
    def _init():
        acc_ref[...] = jnp.zeros_like(acc_ref)

    vacc = jnp.zeros((HBT, L), jnp.float32)
    for w in range(W):
        a = cs_ref[0, :, pl.ds(w * L, L)]
        p = csp_ref[0, :, w, :]
        m = m_ref[0, w, :]
        a = jnp.where(jnp.isnan(a), 0.0, a)
        d = a - jnp.where(m > 0.0, p, 0.0)
        vacc = vacc + d * d
    acc_ref[...] += vacc

    @pl.when((b == B - 1) & (k == NHT - 1))
    def _fin():
        out_ref[0, 0] = jnp.sum(acc_ref[...])


def _tc_call(cs, cs_p, m):
    return pl.pallas_call(
        _tc_body,
        grid=(B, NHT),
        in_specs=[
            pl.BlockSpec((1, HBT, WL), lambda b, k: (b, k, 0)),
            pl.BlockSpec((1, HBT, W, L), lambda b, k: (b, k, 0, 0)),
            pl.BlockSpec((1, W, L), lambda b, k: (b, 0, 0)),
        ],
        out_specs=pl.BlockSpec(memory_space=pltpu.SMEM),
        out_shape=jax.ShapeDtypeStruct((1, 1), jnp.float32),
        scratch_shapes=[pltpu.VMEM((HBT, L), jnp.float32)],
    )(cs, cs_p, m)


def _sc_body(cs_hbm, csp_hbm, m_hbm, out_hbm,
             mbuf, a0, a1, p0, p1, obuf, sa0, sp0, sa1, sp1):
    c = lax.axis_index("c")
    s = lax.axis_index("s")
    wid = s * NC + c
    b = wid // NQ
    q = wid % NQ
    qoff = q * CH
    w0 = q * WQ

    pltpu.sync_copy(m_hbm.at[b, pl.ds(w0, WQ), :], mbuf)

    def _binm(i, carry):
        w = i >> 4
        g = i & 15
        m = mbuf[w, pl.ds(g * LANES, LANES)]
        mbuf[w, pl.ds(g * LANES, LANES)] = jnp.where(m > 0.0, 1.0, 0.0)
        return carry
    lax.fori_loop(0, NV, _binm, 0)

    pltpu.async_copy(cs_hbm.at[b, HS, pl.ds(qoff, CH)], a0, sa0)
    pltpu.async_copy(csp_hbm.at[b, HS, pl.ds(w0, WQ), :], p0, sp0)
    pltpu.async_copy(cs_hbm.at[b, HS + 1, pl.ds(qoff, CH)], a1, sa1)
    pltpu.async_copy(csp_hbm.at[b, HS + 1, pl.ds(w0, WQ), :], p1, sp1)

    def _chunk(abuf, pbuf, acc):
        def _inner(i, acc):
            w = i >> 4
            g = i & 15
            a = abuf[pl.ds(i * LANES, LANES)]
            p = pbuf[w, pl.ds(g * LANES, LANES)]
            m = mbuf[w, pl.ds(g * LANES, LANES)]
            a = jnp.where(jnp.isnan(a), 0.0, a)
            d = a - p * m
            return acc + d * d
        return lax.fori_loop(0, NV, _inner, acc, unroll=8)

    def _outer(k, acc):
        h0 = HS + 2 * k
        pltpu.make_async_copy(cs_hbm.at[b, h0, pl.ds(qoff, CH)], a0, sa0).wait()
        pltpu.make_async_copy(csp_hbm.at[b, h0, pl.ds(w0, WQ), :], p0, sp0).wait()
        acc = _chunk(a0, p0, acc)

        @pl.when(h0 + 2 < H)
        def _():
            pltpu.async_copy(cs_hbm.at[b, h0 + 2, pl.ds(qoff, CH)], a0, sa0)
            pltpu.async_copy(csp_hbm.at[b, h0 + 2, pl.ds(w0, WQ), :], p0, sp0)

        pltpu.make_async_copy(cs_hbm.at[b, h0 + 1, pl.ds(qoff, CH)], a1, sa1).wait()
        pltpu.make_async_copy(csp_hbm.at[b, h0 + 1, pl.ds(w0, WQ), :], p1, sp1).wait()
        acc = _chunk(a1, p1, acc)

        @pl.when(h0 + 3 < H)
        def _():
            pltpu.async_copy(cs_hbm.at[b, h0 + 3, pl.ds(qoff, CH)], a1, sa1)
            pltpu.async_copy(csp_hbm.at[b, h0 + 3, pl.ds(w0, WQ), :], p1, sp1)
        return acc

    acc = lax.fori_loop(0, (H - HS) // 2, _outer, jnp.zeros((LANES,), jnp.float32))
    obuf[...] = acc
    pltpu.sync_copy(obuf, out_hbm.at[wid])


_mesh = plsc.VectorSubcoreMesh(core_axis_name="c", subcore_axis_name="s")

_sc_call = pl.kernel(
    _sc_body,
    out_type=jax.ShapeDtypeStruct((NW, LANES), jnp.float32),
    mesh=_mesh,
    scratch_types=[
        pltpu.VMEM((WQ, L), jnp.float32),
        pltpu.VMEM((CH,), jnp.float32),
        pltpu.VMEM((CH,), jnp.float32),
        pltpu.VMEM((WQ, L), jnp.float32),
        pltpu.VMEM((WQ, L), jnp.float32),
        pltpu.VMEM((LANES,), jnp.float32),
        pltpu.SemaphoreType.DMA,
        pltpu.SemaphoreType.DMA,
        pltpu.SemaphoreType.DMA,
        pltpu.SemaphoreType.DMA,
    ],
)


def kernel(cs, cs_p, overpass_mask):
    sc_partials = _sc_call(cs, cs_p, overpass_mask)
    tc_sum = _tc_call(cs, cs_p, overpass_mask)
    total = tc_sum[0, 0] + jnp.sum(sc_partials)
    return total / jnp.float32(B * H * WL)


# speedup vs baseline: 1.2439x; 1.0422x over previous
import jax
import jax.numpy as jnp
from jax import lax
from jax.experimental import pallas as pl
from jax.experimental.pallas import tpu as pltpu
from jax.experimental.pallas import tpu_sc as plsc

B, H, W, L = 8, 90, 256, 256
WL = W * L
HS = 48
NC, NS, LANES = 2, 16, 16
NW = NC * NS
NQ = NW // B
CH = WL // NQ
WQ = W // NQ
NV = CH // LANES
HBT = 16
NHT = HS // HBT

SH = 7
CWL = 2048
CWW = CWL // L
NCOL = CH // CWL
NSLAB = NCOL * ((H - HS) // SH)
NG = CWL // LANES


def _tc_body(cs_ref, csp_ref, m_ref, out_ref, acc_ref):
    b = pl.program_id(0)
    k = pl.program_id(1)

    @pl.when((b == 0) & (k == 0))
    def _init():
        acc_ref[...] = jnp.zeros_like(acc_ref)

    vacc = jnp.zeros((HBT, L), jnp.float32)
    for w in range(W):
        a = cs_ref[0, :, pl.ds(w * L, L)]
        p = csp_ref[0, :, w, :]
        m = m_ref[0, w, :]
        a = jnp.where(jnp.isnan(a), 0.0, a)
        d = a - jnp.where(m > 0.0, p, 0.0)
        vacc = vacc + d * d
    acc_ref[...] += vacc

    @pl.when((b == B - 1) & (k == NHT - 1))
    def _fin():
        out_ref[0, 0] = jnp.sum(acc_ref[...])


def _tc_call(cs, cs_p, m):
    return pl.pallas_call(
        _tc_body,
        grid=(B, NHT),
        in_specs=[
            pl.BlockSpec((1, HBT, WL), lambda b, k: (b, k, 0)),
            pl.BlockSpec((1, HBT, W, L), lambda b, k: (b, k, 0, 0)),
            pl.BlockSpec((1, W, L), lambda b, k: (b, 0, 0)),
        ],
        out_specs=pl.BlockSpec(memory_space=pltpu.SMEM),
        out_shape=jax.ShapeDtypeStruct((1, 1), jnp.float32),
        scratch_shapes=[pltpu.VMEM((HBT, L), jnp.float32)],
    )(cs, cs_p, m)


def _sc_body(cs_hbm, csp_hbm, m_hbm, out_hbm,
             mbuf, a0, a1, p0, p1, obuf, sa0, sp0, sa1, sp1):
    c = lax.axis_index("c")
    s = lax.axis_index("s")
    wid = s * NC + c
    b = wid // NQ
    q = wid % NQ
    qoff = q * CH
    w0 = q * WQ

    pltpu.sync_copy(m_hbm.at[b, pl.ds(w0, WQ), :], mbuf)

    def _binm(i, carry):
        w = i >> 4
        g = i & 15
        m = mbuf[w, pl.ds(g * LANES, LANES)]
        mbuf[w, pl.ds(g * LANES, LANES)] = jnp.where(m > 0.0, 1.0, 0.0)
        return carry
    lax.fori_loop(0, NV, _binm, 0)

    def _slab_params(t):
        col = t // (NSLAB // NCOL)
        hs = t % (NSLAB // NCOL)
        h0 = HS + hs * SH
        return col, h0

    def _fire(t, abuf, pbuf, sa, sp):
        col, h0 = _slab_params(t)
        for h in range(SH):
            pltpu.async_copy(
                cs_hbm.at[b, h0 + h, pl.ds(qoff + col * CWL, CWL)],
                abuf.at[pl.ds(h * CWL, CWL)], sa)
            pltpu.async_copy(
                csp_hbm.at[b, h0 + h, pl.ds(w0 + col * CWW, CWW), :],
                pbuf.at[pl.ds(h * CWW, CWW), :], sp)

    def _drain(t, abuf, pbuf, sa, sp):
        col, h0 = _slab_params(t)
        for h in range(SH):
            pltpu.make_async_copy(
                cs_hbm.at[b, h0 + h, pl.ds(qoff + col * CWL, CWL)],
                abuf.at[pl.ds(h * CWL, CWL)], sa).wait()
            pltpu.make_async_copy(
                csp_hbm.at[b, h0 + h, pl.ds(w0 + col * CWW, CWW), :],
                pbuf.at[pl.ds(h * CWW, CWW), :], sp).wait()

    _fire(0, a0, p0, sa0, sp0)
    _fire(1, a1, p1, sa1, sp1)

    def _slab_sum(abuf, pbuf, col, acc):
        mw0 = col * CWW

        def _inner(g, acc):
            gw = g >> 4
            gl = (g & 15) * LANES
            m = mbuf[mw0 + gw, pl.ds(gl, LANES)]
            for h in range(SH):
                a = abuf[pl.ds(h * CWL + g * LANES, LANES)]
                p = pbuf[h * CWW + gw, pl.ds(gl, LANES)]
                a = jnp.where(jnp.isnan(a), 0.0, a)
                d = a - p * m
                acc = acc + d * d
            return acc
        return lax.fori_loop(0, NG, _inner, acc, unroll=2)

    def _outer(k, acc):
        t0 = 2 * k
        _drain(t0, a0, p0, sa0, sp0)
        col, _ = _slab_params(t0)
        acc = _slab_sum(a0, p0, col, acc)

        @pl.when(t0 + 2 < NSLAB)
        def _():
            _fire(t0 + 2, a0, p0, sa0, sp0)

        _drain(t0 + 1, a1, p1, sa1, sp1)
        col1, _ = _slab_params(t0 + 1)
        acc = _slab_sum(a1, p1, col1, acc)

        @pl.when(t0 + 3 < NSLAB)
        def _():
            _fire(t0 + 3, a1, p1, sa1, sp1)
        return acc

    acc = lax.fori_loop(0, NSLAB // 2, _outer, jnp.zeros((LANES,), jnp.float32))
    obuf[...] = acc
    pltpu.sync_copy(obuf, out_hbm.at[wid])


_mesh = plsc.VectorSubcoreMesh(core_axis_name="c", subcore_axis_name="s")

_sc_call = pl.kernel(
    _sc_body,
    out_type=jax.ShapeDtypeStruct((NW, LANES), jnp.float32),
    mesh=_mesh,
    scratch_types=[
        pltpu.VMEM((WQ, L), jnp.float32),
        pltpu.VMEM((SH * CWL,), jnp.float32),
        pltpu.VMEM((SH * CWL,), jnp.float32),
        pltpu.VMEM((SH * CWW, L), jnp.float32),
        pltpu.VMEM((SH * CWW, L), jnp.float32),
        pltpu.VMEM((LANES,), jnp.float32),
        pltpu.SemaphoreType.DMA,
        pltpu.SemaphoreType.DMA,
        pltpu.SemaphoreType.DMA,
        pltpu.SemaphoreType.DMA,
    ],
)


def kernel(cs, cs_p, overpass_mask):
    sc_partials = _sc_call(cs, cs_p, overpass_mask)
    tc_sum = _tc_call(cs, cs_p, overpass_mask)
    total = tc_sum[0, 0] + jnp.sum(sc_partials)
    return total / jnp.float32(B * H * WL)
